# baseline (speedup 1.0000x reference)
import functools

import jax
import jax.numpy as jnp
from jax import lax
from jax.experimental import pallas as pl
from jax.experimental.pallas import tpu as pltpu
from jax.experimental.pallas import tpu_sc as plsc

N = 100000
E = 1600000
NPAD = 102400
EB = 2560
NBLK = E // EB
GC = 2000
NW = 32
H = 128
NB = 2048


def _sc_gather(x16, idx_all):
    per_w = (2 * E) // NW
    n_chunks = per_w // GC
    mesh = plsc.VectorSubcoreMesh(core_axis_name="c", subcore_axis_name="s")

    @functools.partial(
        pl.kernel,
        out_type=jax.ShapeDtypeStruct((2 * E, 16), jnp.float32),
        mesh=mesh,
        scratch_types=[
            pltpu.VMEM((GC,), jnp.int32),
            pltpu.VMEM((GC, 16), jnp.float32),
            pltpu.SemaphoreType.DMA,
        ],
        compiler_params=pltpu.CompilerParams(use_tc_tiling_on_sc=False),
    )
    def gather_k(x_hbm, idx_hbm, out_hbm, idx_v, rows_v, sem):
        wid = lax.axis_index("s") * 2 + lax.axis_index("c")
        base = wid * per_w

        def chunk(c, carry):
            off = base + c * GC
            pltpu.sync_copy(idx_hbm.at[pl.ds(off, GC)], idx_v)
            pltpu.async_copy(x_hbm.at[idx_v], rows_v, sem).wait()
            pltpu.sync_copy(rows_v, out_hbm.at[pl.ds(off, GC)])
            return carry

        lax.fori_loop(0, n_chunks, chunk, 0)

    return gather_k(x16, idx_all)


def _edge_mlp_body(xi_ref, xj_ref, w1c_ref, w1b_ref, w2t_ref, b1_ref, b2_ref, out_ref):
    xi = xi_ref[...]
    xj = xj_ref[...]
    dn = (((1,), (1,)), ((), ()))
    pre = (
        lax.dot_general(w1c_ref[...], xi, dn, preferred_element_type=jnp.float32)
        + lax.dot_general(w1b_ref[...], xj, dn, preferred_element_type=jnp.float32)
        + b1_ref[...]
    )
    h = jax.nn.relu(pre)
    m = lax.dot_general(
        w2t_ref[...], h, (((1,), (0,)), ((), ())),
        preferred_element_type=jnp.float32,
    )
    out_ref[0] = jax.nn.relu(m + b2_ref[...])


def _tc_edge_mlp(g, w1c_t, w1b_t, w2_t, b1c, b2c):
    return pl.pallas_call(
        _edge_mlp_body,
        grid=(NBLK,),
        in_specs=[
            pl.BlockSpec((EB, 16), lambda i: (NBLK + i, 0)),
            pl.BlockSpec((EB, 16), lambda i: (i, 0)),
            pl.BlockSpec((H, 16), lambda i: (0, 0)),
            pl.BlockSpec((H, 16), lambda i: (0, 0)),
            pl.BlockSpec((H, H), lambda i: (0, 0)),
            pl.BlockSpec((H, 1), lambda i: (0, 0)),
            pl.BlockSpec((H, 1), lambda i: (0, 0)),
        ],
        out_specs=pl.BlockSpec((1, H, EB), lambda i: (i, 0, 0)),
        out_shape=jax.ShapeDtypeStruct((NBLK, H, EB), jnp.float32),
    )(g, g, w1c_t, w1b_t, w2_t, b1c, b2c)


def _sc_segmax(m3, dst):
    n_vreg = EB // 16
    mesh = plsc.VectorSubcoreMesh(core_axis_name="c", subcore_axis_name="s")

    @functools.partial(
        pl.kernel,
        out_type=jax.ShapeDtypeStruct((H, NPAD), jnp.float32),
        mesh=mesh,
        scratch_types=[
            pltpu.VMEM((NPAD,), jnp.float32),
            pltpu.VMEM((EB,), jnp.float32),
            pltpu.VMEM((EB,), jnp.int32),
        ],
        compiler_params=pltpu.CompilerParams(needs_layout_passes=False),
    )
    def segmax_k(m_hbm, dst_hbm, out_hbm, agg_v, m_v, dst_v):
        wid = lax.axis_index("s") * 2 + lax.axis_index("c")

        for r in range(H // NW):
            ch = wid * (H // NW) + r

            def zero(i, carry):
                agg_v[pl.ds(i * 16, 16)] = jnp.zeros((16,), jnp.float32)
                return carry

            lax.fori_loop(0, NPAD // 16, zero, 0)

            def block(blk, carry):
                pltpu.sync_copy(m_hbm.at[blk, ch], m_v)
                pltpu.sync_copy(dst_hbm.at[pl.ds(blk * EB, EB)], dst_v)

                def vreg(j, c2):
                    idx = dst_v[pl.ds(j * 16, 16)]
                    vals = m_v[pl.ds(j * 16, 16)]
                    cnt, _last = plsc.scan_count(idx)
                    unique = jnp.all(cnt == 1)

                    @pl.when(unique)
                    def _fast():
                        old = plsc.load_gather(agg_v, [idx])
                        plsc.store_scatter(agg_v, [idx], jnp.maximum(old, vals))

                    @pl.when(jnp.logical_not(unique))
                    def _slow():
                        for k in range(1, 17):
                            msk = cnt == k
                            old = plsc.load_gather(agg_v, [idx], mask=msk)
                            new = jnp.maximum(old, vals)
                            plsc.store_scatter(agg_v, [idx], new, mask=msk)

                    return c2

                lax.fori_loop(0, n_vreg, vreg, 0)
                return carry

            lax.fori_loop(0, NBLK, block, 0)
            pltpu.sync_copy(agg_v, out_hbm.at[ch])

    return segmax_k(m3, dst)


def _tail_body(agg_ref, pos_ref, we_ref, be_ref, wd1_ref, bd1_ref, wd2_ref, bd2_ref, out_ref):
    a = agg_ref[...]
    h = we_ref[...] @ a + be_ref[...]
    q = jax.nn.relu(wd1_ref[...] @ h + bd1_ref[...])
    d = wd2_ref[...] @ q + bd2_ref[...]
    out_ref[...] = pos_ref[...] + 0.1 * jnp.tanh(d)


def _tc_tail(agg_t, pos_t, we_t, bec, wd1_t, bd1c, wd2_t, bd2c):
    return pl.pallas_call(
        _tail_body,
        grid=(NPAD // NB,),
        in_specs=[
            pl.BlockSpec((H, NB), lambda i: (0, i)),
            pl.BlockSpec((8, NB), lambda i: (0, i)),
            pl.BlockSpec((H, H), lambda i: (0, 0)),
            pl.BlockSpec((H, 1), lambda i: (0, 0)),
            pl.BlockSpec((H, H), lambda i: (0, 0)),
            pl.BlockSpec((H, 1), lambda i: (0, 0)),
            pl.BlockSpec((8, H), lambda i: (0, 0)),
            pl.BlockSpec((8, 1), lambda i: (0, 0)),
        ],
        out_specs=pl.BlockSpec((8, NB), lambda i: (0, i)),
        out_shape=jax.ShapeDtypeStruct((8, NPAD), jnp.float32),
    )(agg_t, pos_t, we_t, bec, wd1_t, bd1c, wd2_t, bd2c)


def kernel(x, pos, edge_index, W1, b1, W2, b2, We, be, Wd1, bd1, Wd2, bd2):
    f32 = jnp.float32
    x16 = jnp.zeros((N, 16), f32).at[:, :3].set(x)
    idx_all = edge_index.reshape(-1)

    w1a, w1b = W1[:3], W1[3:6]
    w1c_t = jnp.zeros((H, 16), f32).at[:, :3].set((w1a - w1b).T)
    w1b_t = jnp.zeros((H, 16), f32).at[:, :3].set(w1b.T)
    w2_t = W2.T
    b1c = b1[:, None]
    b2c = b2[:, None]

    g = _sc_gather(x16, idx_all)
    m3 = _tc_edge_mlp(g, w1c_t, w1b_t, w2_t, b1c, b2c)
    agg_t = _sc_segmax(m3, edge_index[1])

    pos_t = jnp.zeros((8, NPAD), f32).at[:3, :N].set(pos.T)
    wd2_t = jnp.zeros((8, H), f32).at[:3].set(Wd2.T)
    bd2c = jnp.zeros((8, 1), f32).at[:3, 0].set(bd2)
    out_t = _tc_tail(agg_t, pos_t, We.T, be[:, None], Wd1.T, bd1[:, None], wd2_t, bd2c)
    return out_t[:3, :N].T


# speedup vs baseline: 2.8920x; 2.8920x over previous
import functools

import jax
import jax.numpy as jnp
from jax import lax
from jax.experimental import pallas as pl
from jax.experimental.pallas import tpu as pltpu
from jax.experimental.pallas import tpu_sc as plsc

N = 100000
E = 1600000
NPAD = 102400
EB = 2560
NBLK = E // EB
GC = 2000
NW = 32
H = 128
NB = 2048


def _sc_gather(x16, idx_all):
    per_w = (2 * E) // NW
    n_chunks = per_w // GC
    mesh = plsc.VectorSubcoreMesh(core_axis_name="c", subcore_axis_name="s")

    @functools.partial(
        pl.kernel,
        out_type=jax.ShapeDtypeStruct((2 * E, 16), jnp.float32),
        mesh=mesh,
        scratch_types=[
            pltpu.VMEM((GC,), jnp.int32),
            pltpu.VMEM((GC, 16), jnp.float32),
            pltpu.SemaphoreType.DMA,
        ],
        compiler_params=pltpu.CompilerParams(use_tc_tiling_on_sc=False),
    )
    def gather_k(x_hbm, idx_hbm, out_hbm, idx_v, rows_v, sem):
        wid = lax.axis_index("s") * 2 + lax.axis_index("c")
        base = wid * per_w

        def chunk(c, carry):
            off = base + c * GC
            pltpu.sync_copy(idx_hbm.at[pl.ds(off, GC)], idx_v)
            pltpu.async_copy(x_hbm.at[idx_v], rows_v, sem).wait()
            pltpu.sync_copy(rows_v, out_hbm.at[pl.ds(off, GC)])
            return carry

        lax.fori_loop(0, n_chunks, chunk, 0)

    return gather_k(x16, idx_all)


def _edge_mlp_body(xi_ref, xj_ref, w1c_ref, w1b_ref, w2t_ref, b1_ref, b2_ref, out_ref):
    xi = xi_ref[...]
    xj = xj_ref[...]
    dn = (((1,), (1,)), ((), ()))
    pre = (
        lax.dot_general(w1c_ref[...], xi, dn, preferred_element_type=jnp.float32)
        + lax.dot_general(w1b_ref[...], xj, dn, preferred_element_type=jnp.float32)
        + b1_ref[...]
    )
    h = jax.nn.relu(pre).astype(jnp.bfloat16)
    m = lax.dot_general(
        w2t_ref[...], h, (((1,), (0,)), ((), ())),
        preferred_element_type=jnp.float32,
    )
    out_ref[0] = jax.nn.relu(m + b2_ref[...])


def _tc_edge_mlp(g, w1c_t, w1b_t, w2_t, b1c, b2c):
    return pl.pallas_call(
        _edge_mlp_body,
        grid=(NBLK,),
        in_specs=[
            pl.BlockSpec((EB, 16), lambda i: (NBLK + i, 0)),
            pl.BlockSpec((EB, 16), lambda i: (i, 0)),
            pl.BlockSpec((H, 16), lambda i: (0, 0)),
            pl.BlockSpec((H, 16), lambda i: (0, 0)),
            pl.BlockSpec((H, H), lambda i: (0, 0)),
            pl.BlockSpec((H, 1), lambda i: (0, 0)),
            pl.BlockSpec((H, 1), lambda i: (0, 0)),
        ],
        out_specs=pl.BlockSpec((1, H, EB), lambda i: (i, 0, 0)),
        out_shape=jax.ShapeDtypeStruct((NBLK, H, EB), jnp.float32),
    )(g, g, w1c_t, w1b_t, w2_t, b1c, b2c)


def _sc_segmax(m3, dst):
    n_vreg = EB // 16
    UNROLL = 4
    mesh = plsc.VectorSubcoreMesh(core_axis_name="c", subcore_axis_name="s")

    @functools.partial(
        pl.kernel,
        out_type=jax.ShapeDtypeStruct((H, NPAD), jnp.float32),
        mesh=mesh,
        scratch_types=[
            pltpu.VMEM((NPAD,), jnp.float32),
            pltpu.VMEM((EB,), jnp.float32),
            pltpu.VMEM((EB,), jnp.float32),
            pltpu.VMEM((EB,), jnp.int32),
            pltpu.VMEM((EB,), jnp.int32),
            pltpu.SemaphoreType.DMA,
            pltpu.SemaphoreType.DMA,
            pltpu.SemaphoreType.DMA,
            pltpu.SemaphoreType.DMA,
        ],
        compiler_params=pltpu.CompilerParams(needs_layout_passes=False),
    )
    def segmax_k(m_hbm, dst_hbm, out_hbm, agg_v, m_v0, m_v1, dst_v0, dst_v1,
                 sm0, sm1, sd0, sd1):
        wid = lax.axis_index("s") * 2 + lax.axis_index("c")

        def issue(blk, m_v, dst_v, sm, sd, ch):
            pltpu.async_copy(m_hbm.at[blk, ch], m_v, sm)
            pltpu.async_copy(dst_hbm.at[pl.ds(blk * EB, EB)], dst_v, sd)

        def wait(blk, m_v, dst_v, sm, sd, ch):
            pltpu.make_async_copy(m_hbm.at[blk, ch], m_v, sm).wait()
            pltpu.make_async_copy(dst_hbm.at[pl.ds(blk * EB, EB)], dst_v, sd).wait()

        def process(m_v, dst_v):
            def vreg(jj, c2):
                for u in range(UNROLL):
                    j = jj * UNROLL + u
                    idx = dst_v[pl.ds(j * 16, 16)]
                    vals = m_v[pl.ds(j * 16, 16)]
                    old = plsc.load_gather(agg_v, [idx])
                    plsc.store_scatter(agg_v, [idx], jnp.maximum(old, vals))
                return c2

            lax.fori_loop(0, n_vreg // UNROLL, vreg, 0)

            def check(jj, bad):
                for u in range(UNROLL):
                    j = jj * UNROLL + u
                    idx = dst_v[pl.ds(j * 16, 16)]
                    vals = m_v[pl.ds(j * 16, 16)]
                    chk = plsc.load_gather(agg_v, [idx])
                    bad = bad | jnp.where(chk < vals, 1, 0).astype(jnp.int32)
                return bad

            bad = lax.fori_loop(
                0, n_vreg // UNROLL, check, jnp.zeros((16,), jnp.int32)
            )

            @pl.when(jnp.max(bad) > 0)
            def _fixup():
                def fix(j, c3):
                    idx = dst_v[pl.ds(j * 16, 16)]
                    vals = m_v[pl.ds(j * 16, 16)]
                    cnt, _last = plsc.scan_count(idx)

                    @pl.when(jnp.logical_not(jnp.all(cnt == 1)))
                    def _dup():
                        for k in range(1, 17):
                            msk = cnt == k
                            old = plsc.load_gather(agg_v, [idx], mask=msk)
                            new = jnp.maximum(old, vals)
                            plsc.store_scatter(agg_v, [idx], new, mask=msk)

                    return c3

                lax.fori_loop(0, n_vreg, fix, 0)

        for r in range(H // NW):
            ch = wid * (H // NW) + r

            def zero(i, carry):
                agg_v[pl.ds(i * 16, 16)] = jnp.zeros((16,), jnp.float32)
                return carry

            lax.fori_loop(0, NPAD // 16, zero, 0)

            issue(0, m_v0, dst_v0, sm0, sd0, ch)

            def pair(p, carry):
                b0 = 2 * p
                issue(b0 + 1, m_v1, dst_v1, sm1, sd1, ch)
                wait(b0, m_v0, dst_v0, sm0, sd0, ch)
                process(m_v0, dst_v0)
                issue(b0 + 2, m_v0, dst_v0, sm0, sd0, ch)
                wait(b0 + 1, m_v1, dst_v1, sm1, sd1, ch)
                process(m_v1, dst_v1)
                return carry

            lax.fori_loop(0, NBLK // 2, pair, 0)
            wait(NBLK - 1, m_v0, dst_v0, sm0, sd0, ch)
            process(m_v0, dst_v0)

            pltpu.sync_copy(agg_v, out_hbm.at[ch])

    return segmax_k(m3, dst)


def _tail_body(agg_ref, pos_ref, we_ref, be_ref, wd1_ref, bd1_ref, wd2_ref, bd2_ref, out_ref):
    a = agg_ref[...]
    h = we_ref[...] @ a + be_ref[...]
    q = jax.nn.relu(wd1_ref[...] @ h + bd1_ref[...])
    d = wd2_ref[...] @ q + bd2_ref[...]
    out_ref[...] = pos_ref[...] + 0.1 * jnp.tanh(d)


def _tc_tail(agg_t, pos_t, we_t, bec, wd1_t, bd1c, wd2_t, bd2c):
    return pl.pallas_call(
        _tail_body,
        grid=(NPAD // NB,),
        in_specs=[
            pl.BlockSpec((H, NB), lambda i: (0, i)),
            pl.BlockSpec((8, NB), lambda i: (0, i)),
            pl.BlockSpec((H, H), lambda i: (0, 0)),
            pl.BlockSpec((H, 1), lambda i: (0, 0)),
            pl.BlockSpec((H, H), lambda i: (0, 0)),
            pl.BlockSpec((H, 1), lambda i: (0, 0)),
            pl.BlockSpec((8, H), lambda i: (0, 0)),
            pl.BlockSpec((8, 1), lambda i: (0, 0)),
        ],
        out_specs=pl.BlockSpec((8, NB), lambda i: (0, i)),
        out_shape=jax.ShapeDtypeStruct((8, NPAD), jnp.float32),
    )(agg_t, pos_t, we_t, bec, wd1_t, bd1c, wd2_t, bd2c)


def kernel(x, pos, edge_index, W1, b1, W2, b2, We, be, Wd1, bd1, Wd2, bd2):
    f32 = jnp.float32
    x16 = jnp.zeros((N, 16), f32).at[:, :3].set(x)
    idx_all = edge_index.reshape(-1)

    w1a, w1b = W1[:3], W1[3:6]
    w1c_t = jnp.zeros((H, 16), f32).at[:, :3].set((w1a - w1b).T)
    w1b_t = jnp.zeros((H, 16), f32).at[:, :3].set(w1b.T)
    w2_t = W2.T.astype(jnp.bfloat16)
    b1c = b1[:, None]
    b2c = b2[:, None]

    g = _sc_gather(x16, idx_all)
    m3 = _tc_edge_mlp(g, w1c_t, w1b_t, w2_t, b1c, b2c)
    agg_t = _sc_segmax(m3, edge_index[1])

    pos_t = jnp.zeros((8, NPAD), f32).at[:3, :N].set(pos.T)
    wd2_t = jnp.zeros((8, H), f32).at[:3].set(Wd2.T)
    bd2c = jnp.zeros((8, 1), f32).at[:3, 0].set(bd2)
    out_t = _tc_tail(agg_t, pos_t, We.T, be[:, None], Wd1.T, bd1[:, None], wd2_t, bd2c)
    return out_t[:3, :N].T
